# baseline (speedup 1.0000x reference)
import jax
import jax.numpy as jnp
from jax.experimental import pallas as pl
from jax.experimental.pallas import tpu as pltpu

BM = 512
BN = 2048
WCHUNK = 32


def _linear_kernel(x_ref, w_hbm, b_ref, o_ref, w_a, w_b, w_stage, w_sem):
    j = pl.program_id(0)
    i = pl.program_id(1)
    half = w_a.shape[0]
    n_chunks = half // WCHUNK

    def _start(c, row_base):
        pltpu.make_async_copy(
            w_hbm.at[pl.ds(row_base + c * WCHUNK, WCHUNK), :],
            w_stage.at[c % 2],
            w_sem.at[c % 2]).start()

    def _wait_pack(c, dst, row_base):
        pltpu.make_async_copy(
            w_hbm.at[pl.ds(row_base + c * WCHUNK, WCHUNK), :],
            w_stage.at[c % 2], w_sem.at[c % 2]).wait()
        dst[pl.ds(c * WCHUNK, WCHUNK), :] = (
            w_stage[c % 2].astype(jnp.bfloat16))

    @pl.when((j == 0) & (i == 0))
    def _prologue():
        _start(0, 0)
        _start(1, 0)
        for c in range(n_chunks):
            _wait_pack(c, w_a, 0)
            if c + 2 < n_chunks:
                _start(c + 2, 0)
        _start(0, half)
        _start(1, half)

    for jj in (0, 1):
        @pl.when(j == jj)
        def _dot(jj=jj):
            wref = w_a if jj == 0 else w_b
            acc = jax.lax.dot_general(
                x_ref[...], wref[...],
                dimension_numbers=(((1,), (1,)), ((), ())),
                preferred_element_type=jnp.float32,
            )
            o_ref[...] = acc + b_ref[...]

    @pl.when(j == 0)
    def _prefetch_second_half():
        for cc in range(4):
            c = i * 4 + cc
            _wait_pack(c, w_b, half)

            @pl.when(c + 2 < n_chunks)
            def _issue_next(c=c):
                _start(c + 2, half)


def kernel(input, weight, bias):
    M, K = input.shape
    N = weight.shape[0]
    grid = (2, M // BM)
    return pl.pallas_call(
        _linear_kernel,
        grid=grid,
        in_specs=[
            pl.BlockSpec((BM, K), lambda j, i: (i, 0)),
            pl.BlockSpec(memory_space=pl.ANY),
            pl.BlockSpec((1, BN), lambda j, i: (0, j)),
        ],
        out_specs=pl.BlockSpec((BM, BN), lambda j, i: (i, j)),
        out_shape=jax.ShapeDtypeStruct((M, N), jnp.float32),
        scratch_shapes=[
            pltpu.VMEM((N // 2, K), jnp.bfloat16),
            pltpu.VMEM((N // 2, K), jnp.bfloat16),
            pltpu.VMEM((2, WCHUNK, K), jnp.float32),
            pltpu.SemaphoreType.DMA((2,)),
        ],
        compiler_params=pltpu.CompilerParams(
            dimension_semantics=("arbitrary", "arbitrary"),
            vmem_limit_bytes=60000 * 1024,
        ),
        name="safe_linear",
    )(input, weight, bias.reshape(1, N))


# speedup vs baseline: 1.1482x; 1.1482x over previous
import jax
import jax.numpy as jnp
from jax.experimental import pallas as pl
from jax.experimental.pallas import tpu as pltpu

BM = 512
BN = 2048
KQ = 1024


def _linear_kernel(x_ref, w_hbm, b_ref, o_ref, w_vmem, w_sem):
    j = pl.program_id(0)
    K = x_ref.shape[1]
    n_q = K // KQ

    def _cp(q):
        return pltpu.make_async_copy(
            w_hbm.at[pl.ds(j * BN, BN), pl.ds(q * KQ, KQ)],
            w_vmem.at[:, pl.ds(q * KQ, KQ)],
            w_sem.at[q])

    def _dims():
        return (((1,), (1,)), ((), ()))

    @pl.when(pl.program_id(1) == 0)
    def _transition():
        for q in range(n_q):
            _cp(q).start()
        for q in range(n_q):
            _cp(q).wait()
            part = jax.lax.dot_general(
                x_ref[:, q * KQ:(q + 1) * KQ],
                w_vmem[:, q * KQ:(q + 1) * KQ],
                dimension_numbers=_dims(),
                preferred_element_type=jnp.float32,
            )
            if q == 0:
                o_ref[...] = part + b_ref[...]
            else:
                o_ref[...] += part

    @pl.when(pl.program_id(1) > 0)
    def _steady():
        acc = jax.lax.dot_general(
            x_ref[...], w_vmem[...],
            dimension_numbers=_dims(),
            preferred_element_type=jnp.float32,
        )
        o_ref[...] = acc + b_ref[...]


def kernel(input, weight, bias):
    M, K = input.shape
    N = weight.shape[0]
    grid = (N // BN, M // BM)
    return pl.pallas_call(
        _linear_kernel,
        grid=grid,
        in_specs=[
            pl.BlockSpec((BM, K), lambda j, i: (i, 0)),
            pl.BlockSpec(memory_space=pl.ANY),
            pl.BlockSpec((1, BN), lambda j, i: (0, j)),
        ],
        out_specs=pl.BlockSpec((BM, BN), lambda j, i: (i, j)),
        out_shape=jax.ShapeDtypeStruct((M, N), jnp.float32),
        scratch_shapes=[
            pltpu.VMEM((BN, K), jnp.float32),
            pltpu.SemaphoreType.DMA((4,)),
        ],
        compiler_params=pltpu.CompilerParams(
            dimension_semantics=("arbitrary", "arbitrary"),
            vmem_limit_bytes=60000 * 1024,
        ),
        name="safe_linear",
    )(input, weight, bias.reshape(1, N))


# speedup vs baseline: 1.1518x; 1.0031x over previous
import jax
import jax.numpy as jnp
from jax.experimental import pallas as pl
from jax.experimental.pallas import tpu as pltpu

BM = 512
BN = 2048
KQ = 512


def _linear_kernel(x_ref, w_hbm, b_ref, o_ref, w_vmem, w_sem):
    j = pl.program_id(0)
    K = x_ref.shape[1]
    n_q = K // KQ

    def _cp(q):
        return pltpu.make_async_copy(
            w_hbm.at[pl.ds(j * BN, BN), pl.ds(q * KQ, KQ)],
            w_vmem.at[:, pl.ds(q * KQ, KQ)],
            w_sem.at[q])

    def _dims():
        return (((1,), (1,)), ((), ()))

    @pl.when(pl.program_id(1) == 0)
    def _transition():
        for q in range(n_q):
            _cp(q).start()
        for q in range(n_q):
            _cp(q).wait()
            part = jax.lax.dot_general(
                x_ref[:, q * KQ:(q + 1) * KQ],
                w_vmem[:, q * KQ:(q + 1) * KQ],
                dimension_numbers=_dims(),
                preferred_element_type=jnp.float32,
            )
            if q == 0:
                o_ref[...] = part + b_ref[...]
            else:
                o_ref[...] += part

    @pl.when(pl.program_id(1) > 0)
    def _steady():
        acc = jax.lax.dot_general(
            x_ref[...], w_vmem[...],
            dimension_numbers=_dims(),
            preferred_element_type=jnp.float32,
        )
        o_ref[...] = acc + b_ref[...]


def kernel(input, weight, bias):
    M, K = input.shape
    N = weight.shape[0]
    grid = (N // BN, M // BM)
    return pl.pallas_call(
        _linear_kernel,
        grid=grid,
        in_specs=[
            pl.BlockSpec((BM, K), lambda j, i: (i, 0)),
            pl.BlockSpec(memory_space=pl.ANY),
            pl.BlockSpec((1, BN), lambda j, i: (0, j)),
        ],
        out_specs=pl.BlockSpec((BM, BN), lambda j, i: (i, j)),
        out_shape=jax.ShapeDtypeStruct((M, N), jnp.float32),
        scratch_shapes=[
            pltpu.VMEM((BN, K), jnp.float32),
            pltpu.SemaphoreType.DMA((8,)),
        ],
        compiler_params=pltpu.CompilerParams(
            dimension_semantics=("arbitrary", "arbitrary"),
            vmem_limit_bytes=60000 * 1024,
        ),
        name="safe_linear",
    )(input, weight, bias.reshape(1, N))
